# baseline (speedup 1.0000x reference)
import jax
import jax.numpy as jnp
from jax import lax
from jax.experimental import pallas as pl
from jax.experimental.pallas import tpu as pltpu
from jax.experimental.pallas import tpu_sc as plsc

B, L = 1024, 50
TOK = B * L
V2 = 100002
D = 32
NCONT = 4
NC, NS = 2, 16
NW = NC * NS
TOKW = TOK // NW
T = 80
NCH = TOKW // T
FMAX = 8


def _sc_body(qoe_i, ch_i, fu_i, qoe_x, ch_x, fu_x, qoe_t, ch_t, fu_t,
             wq, bq, wc, bc, wf, bf, out_q, out_c, out_f,
             idxv, idx2v, rowb0, rowb1, xv, contv0, contv1, wv, bv,
             semg, semw0, semw1):
    cid = lax.axis_index("c")
    sid = lax.axis_index("s")
    wid = sid * NC + cid
    tok_base = wid * TOKW
    iota16 = lax.iota(jnp.int32, 16)
    rowb = (rowb0, rowb1)
    contv = (contv0, contv1)
    semw = (semw0, semw1)

    def do_group(F, idx_h, x_h, tab_h, w_h, b_h, out_h):
        pltpu.sync_copy(w_h, wv)
        pltpu.sync_copy(b_h, bv)

        def issue_writes(s, tok0):
            for f in range(F):
                pltpu.async_copy(rowb[s].at[f],
                                 out_h.at[pl.ds(tok0, T), f], semw[s])
            pltpu.async_copy(contv[s],
                             out_h.at[pl.ds(tok0, T), pl.ds(F, NCONT), :],
                             semw[s])

        def drain_writes(s, tok0):
            for f in range(F):
                pltpu.make_async_copy(rowb[s].at[f],
                                      out_h.at[pl.ds(tok0, T), f],
                                      semw[s]).wait()
            pltpu.make_async_copy(contv[s],
                                  out_h.at[pl.ds(tok0, T), pl.ds(F, NCONT), :],
                                  semw[s]).wait()

        def chunk(it, s):
            c = it * 2 + s
            tok0 = tok_base + c * T
            pltpu.sync_copy(idx_h.at[pl.ds(tok0 * F, T * F)],
                            idxv.at[pl.ds(0, T * F)])
            for f in range(F):
                for j in range(T // 16):
                    p = (j * 16 + iota16) * F + f
                    val = plsc.load_gather(idxv, [p])
                    idx2v[f, pl.ds(j * 16, 16)] = val + 1
            @pl.when(it > 0)
            def _():
                drain_writes(s, tok0)
            cps = []
            for f in range(F):
                cps.append(pltpu.async_copy(tab_h.at[f].at[idx2v.at[f]],
                                            rowb[s].at[f], semg))
            pltpu.sync_copy(x_h.at[pl.ds(tok0 * NCONT, T * NCONT)], xv)

            def tloop(t, tc):
                for i in range(NCONT):
                    pi = jnp.broadcast_to(t * NCONT + i, (16,)).astype(jnp.int32)
                    xval = plsc.load_gather(xv, [pi])
                    for h in range(2):
                        sl = pl.ds(h * 16, 16)
                        contv[s][t, i, sl] = xval * wv[i, sl] + bv[i, sl]
                return tc

            lax.fori_loop(0, T, tloop, 0)
            for cp in cps:
                cp.wait()
            issue_writes(s, tok0)

        def pair(it, carry):
            chunk(it, 0)
            chunk(it, 1)
            return carry

        lax.fori_loop(0, NCH // 2, pair, 0)
        for s in (0, 1):
            drain_writes(s, tok_base)

    do_group(6, qoe_i, qoe_x, qoe_t, wq, bq, out_q)
    do_group(8, ch_i, ch_x, ch_t, wc, bc, out_c)
    do_group(6, fu_i, fu_x, fu_t, wf, bf, out_f)


def _tab_tc_body(x_ref, o_ref):
    o_ref[0] = x_ref[0].T


def _make_tab_tc(F):
    vb = 16000
    nv = -(-V2 // vb)
    return pl.pallas_call(
        _tab_tc_body,
        grid=(F, nv),
        in_specs=[pl.BlockSpec((1, D, vb), lambda f, j: (f, 0, j))],
        out_specs=pl.BlockSpec((1, vb, D), lambda f, j: (f, j, 0)),
        out_shape=jax.ShapeDtypeStruct((F, V2, D), jnp.float32),
    )


def _out_tc_body(x_ref, o_ref):
    o_ref[...] = x_ref[...].T


def _make_out_tc(F4):
    C = L * F4 * D

    cb = 3200
    return pl.pallas_call(
        _out_tc_body,
        grid=(B // 128, C // cb),
        in_specs=[pl.BlockSpec((128, cb), lambda g, h: (g, h))],
        out_specs=pl.BlockSpec((cb, 128), lambda g, h: (h, g)),
        out_shape=jax.ShapeDtypeStruct((C, B), jnp.float32),
    )


_mesh = plsc.VectorSubcoreMesh(core_axis_name="c", subcore_axis_name="s")

_kern = pl.kernel(
    _sc_body,
    mesh=_mesh,
    out_type=[
        jax.ShapeDtypeStruct((TOK, 6 + NCONT, D), jnp.float32),
        jax.ShapeDtypeStruct((TOK, 8 + NCONT, D), jnp.float32),
        jax.ShapeDtypeStruct((TOK, 6 + NCONT, D), jnp.float32),
    ],
    scratch_types=[
        pltpu.VMEM((T * FMAX,), jnp.int32),
        pltpu.VMEM((FMAX, T), jnp.int32),
        pltpu.VMEM((FMAX, T, D), jnp.float32),
        pltpu.VMEM((FMAX, T, D), jnp.float32),
        pltpu.VMEM((T * NCONT,), jnp.float32),
        pltpu.VMEM((T, NCONT, D), jnp.float32),
        pltpu.VMEM((T, NCONT, D), jnp.float32),
        pltpu.VMEM((NCONT, D), jnp.float32),
        pltpu.VMEM((NCONT, D), jnp.float32),
        pltpu.SemaphoreType.DMA,
        pltpu.SemaphoreType.DMA,
        pltpu.SemaphoreType.DMA,
    ],
    compiler_params=pltpu.CompilerParams(use_tc_tiling_on_sc=False,
                                         needs_layout_passes=False),
)


def kernel(batch_feature_tensor_pay_QOE_discrete,
           batch_feature_tensor_pay_CHONGHE_discrete,
           batch_feature_tensor_pay_FUFEI_discrete,
           batch_feature_tensor_pay_QOE_continue,
           batch_feature_tensor_pay_CHONGHE_continue,
           batch_feature_tensor_pay_FUFEI_continue,
           QOE_tables, CHONGHE_tables, FUFEI_tables,
           W_QOE, b_QOE, W_CHONGHE, b_CHONGHE, W_FUFEI, b_FUFEI):
    qi = batch_feature_tensor_pay_QOE_discrete.astype(jnp.int32).reshape(-1)
    ci = batch_feature_tensor_pay_CHONGHE_discrete.astype(jnp.int32).reshape(-1)
    fi = batch_feature_tensor_pay_FUFEI_discrete.astype(jnp.int32).reshape(-1)
    qx = batch_feature_tensor_pay_QOE_continue.astype(jnp.float32).reshape(-1)
    cx = batch_feature_tensor_pay_CHONGHE_continue.astype(jnp.float32).reshape(-1)
    fx = batch_feature_tensor_pay_FUFEI_continue.astype(jnp.float32).reshape(-1)
    qt = _make_tab_tc(6)(jnp.transpose(QOE_tables.astype(jnp.float32), (0, 2, 1)))
    ct = _make_tab_tc(8)(jnp.transpose(CHONGHE_tables.astype(jnp.float32), (0, 2, 1)))
    ft = _make_tab_tc(6)(jnp.transpose(FUFEI_tables.astype(jnp.float32), (0, 2, 1)))
    oq, oc, of_ = _kern(qi, ci, fi, qx, cx, fx, qt, ct, ft,
                        W_QOE, b_QOE, W_CHONGHE, b_CHONGHE, W_FUFEI, b_FUFEI)
    oq4 = jnp.transpose(
        _make_out_tc(10)(oq.reshape(B, L * 10 * D)).reshape(L, 10, D, B),
        (3, 0, 1, 2))
    oc4 = jnp.transpose(
        _make_out_tc(12)(oc.reshape(B, L * 12 * D)).reshape(L, 12, D, B),
        (3, 0, 1, 2))
    of4 = jnp.transpose(
        _make_out_tc(10)(of_.reshape(B, L * 10 * D)).reshape(L, 10, D, B),
        (3, 0, 1, 2))
    return (oq4, oc4, of4)


# speedup vs baseline: 2.5043x; 2.5043x over previous
import jax
import jax.numpy as jnp
from jax import lax
from jax.experimental import pallas as pl
from jax.experimental.pallas import tpu as pltpu
from jax.experimental.pallas import tpu_sc as plsc

B, L = 1024, 50
TOK = B * L
V2 = 100002
D = 32
NCONT = 4
NC, NS = 2, 16
NW = NC * NS
TOKW = TOK // NW
T = 80
NCH = TOKW // T

_mesh = plsc.VectorSubcoreMesh(core_axis_name="c", subcore_axis_name="s")


def _make_sc(F):
    F4 = F + NCONT

    def body(idx_h, x_h, tab_h, w_h, b_h, out_h,
             idxv, idx2v, rowb0, rowb1, xv, contv0, contv1, wv, bv,
             semg, semw0, semw1):
        cid = lax.axis_index("c")
        sid = lax.axis_index("s")
        wid = sid * NC + cid
        tok_base = wid * TOKW
        iota16 = lax.iota(jnp.int32, 16)
        rowb = (rowb0, rowb1)
        contv = (contv0, contv1)
        semw = (semw0, semw1)

        pltpu.sync_copy(w_h, wv)
        pltpu.sync_copy(b_h, bv)

        def issue_writes(s, tok0):
            for f in range(F):
                pltpu.async_copy(rowb[s].at[f],
                                 out_h.at[pl.ds(tok0, T), f], semw[s])
            pltpu.async_copy(contv[s],
                             out_h.at[pl.ds(tok0, T), pl.ds(F, NCONT), :],
                             semw[s])

        def drain_writes(s, tok0):
            for f in range(F):
                pltpu.make_async_copy(rowb[s].at[f],
                                      out_h.at[pl.ds(tok0, T), f],
                                      semw[s]).wait()
            pltpu.make_async_copy(contv[s],
                                  out_h.at[pl.ds(tok0, T), pl.ds(F, NCONT), :],
                                  semw[s]).wait()

        def chunk(it, s):
            c = it * 2 + s
            tok0 = tok_base + c * T
            pltpu.sync_copy(idx_h.at[pl.ds(tok0 * F, T * F)], idxv)
            for f in range(F):
                off = f * V2 + 1
                for j in range(T // 16):
                    p = (j * 16 + iota16) * F + f
                    val = plsc.load_gather(idxv, [p])
                    idx2v[f, pl.ds(j * 16, 16)] = val + off
            @pl.when(it > 0)
            def _():
                drain_writes(s, tok0)
            cps = []
            for f in range(F):
                cps.append(pltpu.async_copy(tab_h.at[idx2v.at[f]],
                                            rowb[s].at[f], semg))
            pltpu.sync_copy(x_h.at[pl.ds(tok0 * NCONT, T * NCONT)], xv)

            def tloop(t, tc):
                for i in range(NCONT):
                    pi = jnp.broadcast_to(t * NCONT + i, (16,)).astype(jnp.int32)
                    xval = plsc.load_gather(xv, [pi])
                    for h in range(2):
                        sl = pl.ds(h * 16, 16)
                        contv[s][t, i, sl] = xval * wv[i, sl] + bv[i, sl]
                return tc

            lax.fori_loop(0, T, tloop, 0)
            for cp in cps:
                cp.wait()
            issue_writes(s, tok0)

        def pair(it, carry):
            chunk(it, 0)
            chunk(it, 1)
            return carry

        lax.fori_loop(0, NCH // 2, pair, 0)
        for s in (0, 1):
            drain_writes(s, tok_base)

    return pl.kernel(
        body,
        mesh=_mesh,
        out_type=jax.ShapeDtypeStruct((TOK, F4, D), jnp.float32),
        scratch_types=[
            pltpu.VMEM((T * F,), jnp.int32),
            pltpu.VMEM((F, T), jnp.int32),
            pltpu.VMEM((F, T, D), jnp.float32),
            pltpu.VMEM((F, T, D), jnp.float32),
            pltpu.VMEM((T * NCONT,), jnp.float32),
            pltpu.VMEM((T, NCONT, D), jnp.float32),
            pltpu.VMEM((T, NCONT, D), jnp.float32),
            pltpu.VMEM((NCONT, D), jnp.float32),
            pltpu.VMEM((NCONT, D), jnp.float32),
            pltpu.SemaphoreType.DMA,
            pltpu.SemaphoreType.DMA,
            pltpu.SemaphoreType.DMA,
        ],
        compiler_params=pltpu.CompilerParams(use_tc_tiling_on_sc=False,
                                             needs_layout_passes=False),
    )


_sc6 = _make_sc(6)
_sc8 = _make_sc(8)


def _out_tc_body(x_ref, o_ref):
    o_ref[...] = x_ref[...].T


def _make_out_tc(F4):
    C = L * F4 * D
    cb = 3200
    return pl.pallas_call(
        _out_tc_body,
        grid=(B // 128, C // cb),
        in_specs=[pl.BlockSpec((128, cb), lambda g, h: (g, h))],
        out_specs=pl.BlockSpec((cb, 128), lambda g, h: (h, g)),
        out_shape=jax.ShapeDtypeStruct((C, B), jnp.float32),
    )


def _finish(o, F4):
    return jnp.transpose(
        _make_out_tc(F4)(o.reshape(B, L * F4 * D)).reshape(L, F4, D, B),
        (3, 0, 1, 2))


def kernel(batch_feature_tensor_pay_QOE_discrete,
           batch_feature_tensor_pay_CHONGHE_discrete,
           batch_feature_tensor_pay_FUFEI_discrete,
           batch_feature_tensor_pay_QOE_continue,
           batch_feature_tensor_pay_CHONGHE_continue,
           batch_feature_tensor_pay_FUFEI_continue,
           QOE_tables, CHONGHE_tables, FUFEI_tables,
           W_QOE, b_QOE, W_CHONGHE, b_CHONGHE, W_FUFEI, b_FUFEI):
    qi = batch_feature_tensor_pay_QOE_discrete.astype(jnp.int32).reshape(-1)
    ci = batch_feature_tensor_pay_CHONGHE_discrete.astype(jnp.int32).reshape(-1)
    fi = batch_feature_tensor_pay_FUFEI_discrete.astype(jnp.int32).reshape(-1)
    qx = batch_feature_tensor_pay_QOE_continue.astype(jnp.float32).reshape(-1)
    cx = batch_feature_tensor_pay_CHONGHE_continue.astype(jnp.float32).reshape(-1)
    fx = batch_feature_tensor_pay_FUFEI_continue.astype(jnp.float32).reshape(-1)
    qt = QOE_tables.reshape(6 * V2, D)
    ct = CHONGHE_tables.reshape(8 * V2, D)
    ft = FUFEI_tables.reshape(6 * V2, D)
    oq = _sc6(qi, qx, qt, W_QOE, b_QOE)
    oc = _sc8(ci, cx, ct, W_CHONGHE, b_CHONGHE)
    of_ = _sc6(fi, fx, ft, W_FUFEI, b_FUFEI)
    return (_finish(oq, 10), _finish(oc, 12), _finish(of_, 10))
